# baseline (speedup 1.0000x reference)
import functools

import jax
import jax.numpy as jnp
from jax import lax
from jax.experimental import pallas as pl
from jax.experimental.pallas import tpu as pltpu
from jax.experimental.pallas import tpu_sc as plsc

HEAD_DIM = 128
BLOCK_DIM = 8
N_BLOCKS = HEAD_DIM // BLOCK_DIM
N_CENT = 256
WIDE = N_BLOCKS * N_CENT
TABLE_ELEMS = WIDE * BLOCK_DIM

TILE = 2048

SC_NC = 2
SC_NS = 16
SC_WORKERS = SC_NC * SC_NS
SC_CHUNK_ROWS = 2048


def _block_diag(mats):
    G, a, b = mats.shape
    eye = jnp.eye(G, dtype=mats.dtype)
    return jnp.einsum('gab,gh->gahb', mats, eye).reshape(G * a, G * b)


def _vq_kernel(k_ref, e_ref, c_ref, c2h_ref, idx_ref, idxf_ref):
    x = k_ref[...]
    kshaped = jnp.dot(x, e_ref[...], preferred_element_type=jnp.float32,
                      precision=lax.Precision.DEFAULT)
    cross = jnp.dot(kshaped, c_ref[...], preferred_element_type=jnp.float32,
                    precision=lax.Precision.DEFAULT)
    cols = []
    for b in range(N_BLOCKS):
        sl = slice(b * N_CENT, (b + 1) * N_CENT)
        s = cross[:, sl] - c2h_ref[:, sl]
        ib = jnp.argmax(s, axis=1).astype(jnp.int32)
        cols.append(ib[:, None])
    idx = jnp.concatenate(cols, axis=1)
    idx_ref[...] = idx
    idxf_ref[...] = idx * BLOCK_DIM + lax.broadcasted_iota(
        jnp.int32, (TILE, N_BLOCKS), 1) * (N_CENT * BLOCK_DIM)


def _dequantize_sc(table_flat, idxf_flat):
    b_tot = idxf_flat.shape[0]
    rows_pw = b_tot // SC_WORKERS
    n_chunks = rows_pw // SC_CHUNK_ROWS
    mesh = plsc.VectorSubcoreMesh(core_axis_name="c", subcore_axis_name="s")

    @functools.partial(
        pl.kernel, mesh=mesh,
        out_type=jax.ShapeDtypeStruct((b_tot * BLOCK_DIM,), jnp.float32),
        compiler_params=pltpu.CompilerParams(needs_layout_passes=False),
        scratch_types=[
            pltpu.VMEM((TABLE_ELEMS,), jnp.float32),
            pltpu.VMEM((SC_CHUNK_ROWS,), jnp.int32),
            pltpu.VMEM((SC_CHUNK_ROWS * BLOCK_DIM,), jnp.float32),
        ],
    )
    def dq_k(table_hbm, idxf_hbm, out_hbm, table_v, idx_v, out_v):
        wid = lax.axis_index("s") * SC_NC + lax.axis_index("c")
        pltpu.sync_copy(table_hbm, table_v)
        iota = lax.iota(jnp.int32, 16)
        pair = lax.shift_right_logical(iota, 3)
        lane8 = lax.bitwise_and(iota, 7)

        @pl.loop(0, n_chunks)
        def _chunk(c):
            base_row = wid * rows_pw + c * SC_CHUNK_ROWS
            pltpu.sync_copy(idxf_hbm.at[pl.ds(base_row, SC_CHUNK_ROWS)],
                            idx_v)

            @plsc.parallel_loop(0, SC_CHUNK_ROWS // 2, unroll=8)
            def _v(v):
                r8 = plsc.load_gather(idx_v, [pair + 2 * v])
                vals = plsc.load_gather(table_v, [r8 + lane8])
                out_v[pl.ds(v * 16, 16)] = vals

            pltpu.sync_copy(
                out_v,
                out_hbm.at[pl.ds(base_row * BLOCK_DIM,
                                 SC_CHUNK_ROWS * BLOCK_DIM)])

    return dq_k(table_flat, idxf_flat)


@jax.jit
def kernel(k, E_blocks, centroids, decoded_centroids):
    batch_shape = k.shape[:-1]
    kf = k.reshape(-1, HEAD_DIM).astype(jnp.float32)
    n = kf.shape[0]

    e_bd = _block_diag(jnp.transpose(E_blocks, (0, 2, 1)))
    c_bd = _block_diag(jnp.transpose(centroids, (0, 2, 1)))
    c2h = 0.5 * jnp.sum(centroids * centroids, axis=-1).reshape(1, WIDE)

    grid = (n // TILE,)
    const = lambda i: (0, 0)
    idx, idxf = pl.pallas_call(
        _vq_kernel,
        grid=grid,
        in_specs=[
            pl.BlockSpec((TILE, HEAD_DIM), lambda i: (i, 0)),
            pl.BlockSpec((HEAD_DIM, HEAD_DIM), const),
            pl.BlockSpec((HEAD_DIM, WIDE), const),
            pl.BlockSpec((1, WIDE), const),
        ],
        out_specs=[
            pl.BlockSpec((TILE, N_BLOCKS), lambda i: (i, 0)),
            pl.BlockSpec((TILE, N_BLOCKS), lambda i: (i, 0)),
        ],
        out_shape=[
            jax.ShapeDtypeStruct((n, N_BLOCKS), jnp.int32),
            jax.ShapeDtypeStruct((n, N_BLOCKS), jnp.int32),
        ],
        compiler_params=pltpu.CompilerParams(
            dimension_semantics=("arbitrary",),
        ),
    )(kf, e_bd, c_bd, c2h)

    table_flat = decoded_centroids.reshape(-1)
    khat = _dequantize_sc(table_flat, idxf.reshape(-1))

    return (idx.reshape(*batch_shape, N_BLOCKS),
            khat.reshape(*batch_shape, HEAD_DIM))


# speedup vs baseline: 1.1980x; 1.1980x over previous
import functools

import jax
import jax.numpy as jnp
from jax import lax
from jax.experimental import pallas as pl
from jax.experimental.pallas import tpu as pltpu

HEAD_DIM = 128
BLOCK_DIM = 8
N_BLOCKS = HEAD_DIM // BLOCK_DIM
N_CENT = 256
WIDE = N_BLOCKS * N_CENT

TILE = 2048


def _block_diag(mats):
    G, a, b = mats.shape
    eye = jnp.eye(G, dtype=mats.dtype)
    return jnp.einsum('gab,gh->gahb', mats, eye).reshape(G * a, G * b)


def _vq_kernel(k_ref, e_ref, c_ref, c2h_ref, dc_ref,
               idx_ref, khat_ref):
    x = k_ref[...]
    kshaped = jnp.dot(x, e_ref[...], preferred_element_type=jnp.float32,
                      precision=lax.Precision.DEFAULT)
    cross = jnp.dot(kshaped, c_ref[...], preferred_element_type=jnp.float32,
                    precision=lax.Precision.DEFAULT)
    cols = []
    onehots = []
    for b in range(N_BLOCKS):
        sl = slice(b * N_CENT, (b + 1) * N_CENT)
        s = cross[:, sl] - c2h_ref[:, sl]
        ib = jnp.argmax(s, axis=1).astype(jnp.int32)
        cols.append(ib[:, None])
        onehots.append(
            (lax.broadcasted_iota(jnp.int32, (TILE, N_CENT), 1)
             == ib[:, None]).astype(jnp.bfloat16))
    idx_ref[...] = jnp.concatenate(cols, axis=1)
    onehot = jnp.concatenate(onehots, axis=1)
    khat_ref[...] = jnp.dot(onehot, dc_ref[...],
                            preferred_element_type=jnp.float32,
                            precision=lax.Precision.DEFAULT)


@jax.jit
def kernel(k, E_blocks, centroids, decoded_centroids):
    batch_shape = k.shape[:-1]
    kf = k.reshape(-1, HEAD_DIM).astype(jnp.float32)
    n = kf.shape[0]

    e_bd = _block_diag(jnp.transpose(E_blocks, (0, 2, 1)))
    c_bd = _block_diag(jnp.transpose(centroids, (0, 2, 1)))
    dc_bd = _block_diag(decoded_centroids).astype(jnp.bfloat16)
    c2h = 0.5 * jnp.sum(centroids * centroids, axis=-1).reshape(1, WIDE)

    grid = (n // TILE,)
    const = lambda i: (0, 0)
    idx, khat = pl.pallas_call(
        _vq_kernel,
        grid=grid,
        in_specs=[
            pl.BlockSpec((TILE, HEAD_DIM), lambda i: (i, 0)),
            pl.BlockSpec((HEAD_DIM, HEAD_DIM), const),
            pl.BlockSpec((HEAD_DIM, WIDE), const),
            pl.BlockSpec((1, WIDE), const),
            pl.BlockSpec((WIDE, HEAD_DIM), const),
        ],
        out_specs=[
            pl.BlockSpec((TILE, N_BLOCKS), lambda i: (i, 0)),
            pl.BlockSpec((TILE, HEAD_DIM), lambda i: (i, 0)),
        ],
        out_shape=[
            jax.ShapeDtypeStruct((n, N_BLOCKS), jnp.int32),
            jax.ShapeDtypeStruct((n, HEAD_DIM), jnp.float32),
        ],
        compiler_params=pltpu.CompilerParams(
            dimension_semantics=("arbitrary",),
        ),
    )(kf, e_bd, c_bd, c2h, dc_bd)

    return (idx.reshape(*batch_shape, N_BLOCKS),
            khat.reshape(*batch_shape, HEAD_DIM))
